# Initial kernel scaffold; baseline (speedup 1.0000x reference)
import jax
import jax.numpy as jnp
from jax.experimental import pallas as pl


def kernel(node_feat, time_feat, context_feat, W_q, b_q, W_k, b_k, W_v, b_v, cluster_emb, W_o, b_o, edge_index):
    raise NotImplementedError("write your pallas kernel here")


# speedup vs baseline: 25.0560x; 25.0560x over previous
import functools

import jax
import jax.numpy as jnp
from jax import lax
from jax.experimental import pallas as pl
from jax.experimental.pallas import tpu as pltpu
from jax.experimental.pallas import tpu_sc as plsc

N = 10000
E = 160000
D = 256
C = 8

NC = 2
NS = 16
NW = NC * NS
CHK = 128
NCHUNK = E // CHK
KMAX = (NCHUNK + NW - 1) // NW
RCH = 40
OCH = 2000

EB = 1600
NEB = E // EB
NB = 1000


def _fill(ref, value, n):
    vec = jnp.full((16,), value, dtype=ref.dtype)

    def body(i, _):
        ref[pl.ds(i * 16, 16)] = vec
        return 0

    lax.fori_loop(0, n // 16, body, 0)
    if n % 16:
        ref[pl.ds(n - 16, 16)] = vec


def _fill2d(ref, value, rows, cols):
    vec = jnp.full((16,), value, dtype=ref.dtype)

    def body(i, _):
        r = i // (cols // 16)
        k = i % (cols // 16)
        ref[r, pl.ds(k * 16, 16)] = vec
        return 0

    lax.fori_loop(0, rows * (cols // 16), body, 0)


def _k1_body(nf, wq, wk, bq, qk_out):
    q = lax.dot_general(nf[...], wq[...], (((1,), (1,)), ((), ())),
                        preferred_element_type=jnp.float32) + bq[...]
    qk = lax.dot_general(q, wk[...], (((1,), (0,)), ((), ())),
                         preferred_element_type=jnp.float32)
    qk_out[...] = qk * (D ** -0.5)


def _k1(node_feat, W_q, W_k, b_q):
    return pl.pallas_call(
        _k1_body,
        grid=(N // NB,),
        in_specs=[
            pl.BlockSpec((NB, D), lambda i: (i, 0)),
            pl.BlockSpec((D, D), lambda i: (0, 0)),
            pl.BlockSpec((D, D), lambda i: (0, 0)),
            pl.BlockSpec((1, D), lambda i: (0, 0)),
        ],
        out_specs=pl.BlockSpec((NB, D), lambda i: (i, 0)),
        out_shape=jax.ShapeDtypeStruct((N, D), jnp.float32),
    )(node_feat, W_q, W_k, b_q)


def _k2_kernel(qk_hbm, src_hbm, g_hbm, idx_v, rows_v, sem):
    c = lax.axis_index("c")
    s = lax.axis_index("s")
    w = s * NC + c

    def body(k, _):
        j = w + k * NW

        @pl.when(j < NCHUNK)
        def _():
            off = j * CHK
            pltpu.sync_copy(src_hbm.at[pl.ds(off, CHK)], idx_v)
            pltpu.async_copy(qk_hbm.at[idx_v], rows_v, sem).wait()
            pltpu.sync_copy(rows_v, g_hbm.at[pl.ds(off, CHK)])
        return 0

    lax.fori_loop(0, KMAX, body, 0)


def _k2(qk, src):
    mesh = plsc.VectorSubcoreMesh(core_axis_name="c", subcore_axis_name="s")
    f = pl.kernel(
        _k2_kernel, mesh=mesh,
        out_type=jax.ShapeDtypeStruct((E, D), jnp.float32),
        scratch_types=[
            pltpu.VMEM((CHK,), jnp.int32),
            pltpu.VMEM((CHK, D), jnp.float32),
            pltpu.SemaphoreType.DMA,
        ],
    )
    return f(qk, src)


def _k3_body(t, g, ce, src3, attn_out, seg_out, m_out, m_ref):
    i = pl.program_id(0)
    tb = t[...]
    attn = jnp.sum(g[...] * tb, axis=-1)
    sim = lax.dot_general(tb, ce[...], (((1,), (1,)), ((), ())),
                          preferred_element_type=jnp.float32)
    mx = jnp.max(sim, axis=-1, keepdims=True)
    iota = lax.broadcasted_iota(jnp.int32, (EB, C), 1)
    assign = jnp.min(jnp.where(sim == mx, iota, C), axis=-1)
    attn_out[...] = attn[None, None, :]
    seg_out[...] = (src3[0, 0, :] * C + assign)[None, None, :]
    bm = jnp.max(attn)

    @pl.when(i == 0)
    def _():
        m_ref[0, 0] = bm

    @pl.when(i > 0)
    def _():
        m_ref[0, 0] = jnp.maximum(m_ref[0, 0], bm)

    m_out[0, 0] = m_ref[0, 0]


def _k3(time_feat, G, cluster_emb, src3):
    return pl.pallas_call(
        _k3_body,
        grid=(NEB,),
        in_specs=[
            pl.BlockSpec((EB, D), lambda i: (i, 0)),
            pl.BlockSpec((EB, D), lambda i: (i, 0)),
            pl.BlockSpec((C, D), lambda i: (0, 0)),
            pl.BlockSpec((1, 1, EB), lambda i: (i, 0, 0)),
        ],
        out_specs=[
            pl.BlockSpec((1, 1, EB), lambda i: (i, 0, 0)),
            pl.BlockSpec((1, 1, EB), lambda i: (i, 0, 0)),
            pl.BlockSpec(memory_space=pltpu.SMEM),
        ],
        out_shape=[
            jax.ShapeDtypeStruct((NEB, 1, EB), jnp.float32),
            jax.ShapeDtypeStruct((NEB, 1, EB), jnp.int32),
            jax.ShapeDtypeStruct((1, 1), jnp.float32),
        ],
        scratch_shapes=[pltpu.SMEM((1, 1), jnp.float32)],
    )(time_feat, G, cluster_emb, src3)


def _k3b_body(attn, m, out):
    out[...] = jnp.exp(attn[...] - m[0, 0])


def _k3b(attn3, M):
    return pl.pallas_call(
        _k3b_body,
        grid=(NEB,),
        in_specs=[
            pl.BlockSpec((1, 1, EB), lambda i: (i, 0, 0)),
            pl.BlockSpec(memory_space=pltpu.SMEM),
        ],
        out_specs=pl.BlockSpec((1, 1, EB), lambda i: (i, 0, 0)),
        out_shape=jax.ShapeDtypeStruct((NEB, 1, EB), jnp.float32),
    )(attn3, M)


def _k4_kernel(seg_hbm, expv_hbm, s_out, c_out,
               seg_v, val_v, ones_v, zb_v, s_tab, c_tab):
    c = lax.axis_index("c")
    s = lax.axis_index("s")
    w = s * NC + c
    _fill(ones_v, 1.0, CHK)
    _fill(zb_v, 0.0, OCH)
    nzch = (N * C) // OCH

    def zbody(k, _):
        j = s + k * NS

        @pl.when(j < nzch)
        def _():
            pltpu.sync_copy(zb_v, s_tab.at[pl.ds(j * OCH, OCH)])
            pltpu.sync_copy(zb_v, c_tab.at[pl.ds(j * OCH, OCH)])
        return 0

    lax.fori_loop(0, (nzch + NS - 1) // NS, zbody, 0)
    plsc.subcore_barrier()

    def body(k, _):
        j = w + k * NW

        @pl.when(j < NCHUNK)
        def _():
            off = j * CHK
            pltpu.sync_copy(seg_hbm.at[pl.ds(off, CHK)], seg_v)
            pltpu.sync_copy(expv_hbm.at[pl.ds(off, CHK)], val_v)
            pltpu.sync_copy(val_v, s_tab.at[seg_v], add=True)
            pltpu.sync_copy(ones_v, c_tab.at[seg_v], add=True)
        return 0

    lax.fori_loop(0, KMAX, body, 0)
    plsc.subcore_barrier()

    def obody(k, _):
        j = s + k * NS

        @pl.when(j < nzch)
        def _():
            pltpu.sync_copy(s_tab.at[pl.ds(j * OCH, OCH)], zb_v)
            pltpu.sync_copy(zb_v, s_out.at[pl.ds(c * (N * C) + j * OCH, OCH)])
            pltpu.sync_copy(c_tab.at[pl.ds(j * OCH, OCH)], zb_v)
            pltpu.sync_copy(zb_v, c_out.at[pl.ds(c * (N * C) + j * OCH, OCH)])
        return 0

    lax.fori_loop(0, (nzch + NS - 1) // NS, obody, 0)


def _k4(seg, expv):
    mesh = plsc.VectorSubcoreMesh(core_axis_name="c", subcore_axis_name="s")
    f = pl.kernel(
        _k4_kernel, mesh=mesh,
        out_type=[
            jax.ShapeDtypeStruct((NC * N * C,), jnp.float32),
            jax.ShapeDtypeStruct((NC * N * C,), jnp.float32),
        ],
        scratch_types=[
            pltpu.VMEM((CHK,), jnp.int32),
            pltpu.VMEM((CHK,), jnp.float32),
            pltpu.VMEM((CHK,), jnp.float32),
            pltpu.VMEM((OCH,), jnp.float32),
            pltpu.VMEM_SHARED((N * C,), jnp.float32),
            pltpu.VMEM_SHARED((N * C,), jnp.float32),
        ],
    )
    return f(seg, expv)


def _k5_body(s2, c2, factor_out, w_out):
    s = s2[0] + s2[1]
    cnt = c2[0] + c2[1]
    nonempty = (jnp.max(cnt, axis=0) > 0).astype(jnp.float32)
    n_ne = jnp.sum(nonempty)
    safe_s = jnp.where(s > 0, s, 1.0)
    safe_c = jnp.maximum(cnt, 1.0)
    factor_out[...] = 1.0 / (safe_s * safe_c * n_ne)
    w = jnp.sum(jnp.where(cnt > 0, 1.0 / safe_c, 0.0), axis=1) / n_ne
    w_out[...] = jnp.broadcast_to(w[:, None], (N, C))


def _k5(s2, c2):
    return pl.pallas_call(
        _k5_body,
        out_shape=[
            jax.ShapeDtypeStruct((N, C), jnp.float32),
            jax.ShapeDtypeStruct((N, C), jnp.float32),
        ],
    )(s2, c2)


def _k5b_body(factor, out):
    fb = factor[...]
    rep = jnp.broadcast_to(fb[:, :, None], (NB, C, 128)).reshape(NB, C * 128)
    lane = lax.broadcasted_iota(jnp.int32, (NB, C * 128), 1)
    out[...] = jnp.where(lane % 128 == 0, rep, 0.0)


def _k5b(factor):
    return pl.pallas_call(
        _k5b_body,
        grid=(N // NB,),
        in_specs=[pl.BlockSpec((NB, C), lambda i: (i, 0))],
        out_specs=pl.BlockSpec((NB, C * 128), lambda i: (i, 0)),
        out_shape=jax.ShapeDtypeStruct((N, C * 128), jnp.float32),
    )(factor)


def _k6a_kernel(ftab_hbm, seg_hbm, f_hbm, seg_v, rows_v, sem):
    c = lax.axis_index("c")
    s = lax.axis_index("s")
    w = s * NC + c

    def body(k, _):
        j = w + k * NW

        @pl.when(j < NCHUNK)
        def _():
            off = j * CHK
            pltpu.sync_copy(seg_hbm.at[pl.ds(off, CHK)], seg_v)
            pltpu.async_copy(ftab_hbm.at[seg_v], rows_v, sem).wait()
            pltpu.sync_copy(rows_v, f_hbm.at[pl.ds(off, CHK)])
        return 0

    lax.fori_loop(0, KMAX, body, 0)


def _k6a(ftab, seg):
    mesh = plsc.VectorSubcoreMesh(core_axis_name="c", subcore_axis_name="s")
    f = pl.kernel(
        _k6a_kernel, mesh=mesh,
        out_type=jax.ShapeDtypeStruct((E, 128), jnp.float32),
        scratch_types=[
            pltpu.VMEM((CHK,), jnp.int32),
            pltpu.VMEM((CHK, 128), jnp.float32),
            pltpu.SemaphoreType.DMA,
        ],
    )
    return f(ftab, seg)


def _k6b_body(t, expv, f, w0, w1):
    cvec = expv[0, 0, :][:, None] * f[:, 0:1]
    wt = t[...] * cvec
    w0[...] = wt[:, 0 * QD:1 * QD]
    w1[...] = wt[:, 1 * QD:2 * QD]


def _k6b(time_feat, expv3, f3):
    return pl.pallas_call(
        _k6b_body,
        grid=(NEB,),
        in_specs=[
            pl.BlockSpec((EB, D), lambda i: (i, 0)),
            pl.BlockSpec((1, 1, EB), lambda i: (i, 0, 0)),
            pl.BlockSpec((EB, 128), lambda i: (i, 0)),
        ],
        out_specs=[pl.BlockSpec((EB, QD), lambda i: (i, 0))] * NQ,
        out_shape=[jax.ShapeDtypeStruct((E, QD), jnp.float32)] * NQ,
    )(time_feat, expv3, f3)


QD = D // 2
NQ = 2


def _k7_kernel(wt0, wt1, src_hbm, t_out, idx_v, wt_v, st_v, t_tab):
    wts = (wt0, wt1)
    c = lax.axis_index("c")
    s = lax.axis_index("s")
    w = s * NC + c
    nrch = N // RCH
    for h in range(NQ):
        wt_hbm = wts[h]
        _fill2d(st_v, 0.0, RCH, QD)

        def zbody(k, _):
            j = s + k * NS

            @pl.when(j < nrch)
            def _():
                pltpu.sync_copy(st_v, t_tab.at[pl.ds(j * RCH, RCH)])
            return 0

        lax.fori_loop(0, (nrch + NS - 1) // NS, zbody, 0)
        plsc.subcore_barrier()

        def body(k, _):
            j = w + k * NW

            @pl.when(j < NCHUNK)
            def _():
                off = j * CHK
                pltpu.sync_copy(src_hbm.at[pl.ds(off, CHK)], idx_v)
                pltpu.sync_copy(wt_hbm.at[pl.ds(off, CHK)], wt_v)
                pltpu.sync_copy(wt_v, t_tab.at[idx_v], add=True)
            return 0

        lax.fori_loop(0, KMAX, body, 0)
        plsc.subcore_barrier()

        def obody(k, _):
            j = s + k * NS

            @pl.when(j < nrch)
            def _():
                pltpu.sync_copy(t_tab.at[pl.ds(j * RCH, RCH)], st_v)
                pltpu.sync_copy(st_v, t_out.at[h, c, pl.ds(j * RCH, RCH)])
            return 0

        lax.fori_loop(0, (nrch + NS - 1) // NS, obody, 0)
        plsc.subcore_barrier()


def _k7(wt4, src):
    mesh = plsc.VectorSubcoreMesh(core_axis_name="c", subcore_axis_name="s")
    f = pl.kernel(
        _k7_kernel, mesh=mesh,
        out_type=jax.ShapeDtypeStruct((NQ, NC, N, QD), jnp.float32),
        scratch_types=[
            pltpu.VMEM((CHK,), jnp.int32),
            pltpu.VMEM((CHK, QD), jnp.float32),
            pltpu.VMEM((RCH, QD), jnp.float32),
            pltpu.VMEM_SHARED((N, QD), jnp.float32),
        ],
    )
    return f(*wt4, src)


def _k8_body(tt, w, wv, bv, wo, bo, out):
    t = jnp.concatenate([tt[q, 0] + tt[q, 1] for q in range(NQ)], axis=-1)
    combined = lax.dot_general(t, wv[...], (((1,), (1,)), ((), ())),
                               preferred_element_type=jnp.float32)
    combined = combined + w[:, 0:1] * bv[...]
    o = lax.dot_general(combined, wo[...], (((1,), (1,)), ((), ())),
                        preferred_element_type=jnp.float32) + bo[...]
    out[...] = jnp.maximum(o, 0.0)


def _k8(T4, w, W_v, b_v, W_o, b_o):
    return pl.pallas_call(
        _k8_body,
        grid=(N // NB,),
        in_specs=[
            pl.BlockSpec((NQ, NC, NB, QD), lambda i: (0, 0, i, 0)),
            pl.BlockSpec((NB, C), lambda i: (i, 0)),
            pl.BlockSpec((D, D), lambda i: (0, 0)),
            pl.BlockSpec((1, D), lambda i: (0, 0)),
            pl.BlockSpec((D, D), lambda i: (0, 0)),
            pl.BlockSpec((1, D), lambda i: (0, 0)),
        ],
        out_specs=pl.BlockSpec((NB, D), lambda i: (i, 0)),
        out_shape=jax.ShapeDtypeStruct((N, D), jnp.float32),
    )(T4, w, W_v, b_v, W_o, b_o)


def kernel(node_feat, time_feat, context_feat, W_q, b_q, W_k, b_k, W_v, b_v,
           cluster_emb, W_o, b_o, edge_index):
    src = jnp.asarray(edge_index[0], jnp.int32)
    src3 = src.reshape(NEB, 1, EB)
    b_q2 = b_q.reshape(1, D)
    b_v2 = b_v.reshape(1, D)
    b_o2 = b_o.reshape(1, D)

    qk = _k1(node_feat, W_q, W_k, b_q2)
    G = _k2(qk, src)
    attn3, seg3, M = _k3(time_feat, G, cluster_emb, src3)
    expv3 = _k3b(attn3, M)
    seg = seg3.reshape(E)
    expv = expv3.reshape(E)
    s2, c2 = _k4(seg, expv)
    factor, w = _k5(s2.reshape(NC, N, C), c2.reshape(NC, N, C))
    f16 = _k6a(_k5b(factor).reshape(N * C, 128), seg)
    wt4 = _k6b(time_feat, expv3, f16)
    T4 = _k7(wt4, src)
    return _k8(T4, w, W_v, b_v2, W_o, b_o2)
